# speedup vs baseline: 3.0571x; 3.0571x over previous
import functools

import jax
import jax.numpy as jnp
from jax import lax
from jax.experimental import pallas as pl
from jax.experimental.pallas import tpu as pltpu
from jax.experimental.pallas import tpu_sc as plsc

B = 50000
D = 128
E = 128
NSAMP = 10
NLSH = 5

NC = 2
NS = 16
NW = NC * NS
C = 16
NCHUNK = 98
PW = C * NCHUNK
BP = NW * PW

TB = 512


def _sc_body(feat, nodes, neigh, lsh, self_o, neigh_o, lsh_o,
             nodes_v, neigh_v, lsh_v, srows, nrows, lrows, nmean, lmean,
             sem_s, sem_n, sem_l):
  wid = lax.axis_index("s") * NC + lax.axis_index("c")
  pltpu.sync_copy(nodes.at[pl.ds(wid * PW, PW)], nodes_v)
  pltpu.sync_copy(neigh.at[pl.ds(wid * PW * NSAMP, PW * NSAMP)], neigh_v)
  pltpu.sync_copy(lsh.at[pl.ds(wid * PW * NLSH, PW * NLSH)], lsh_v)

  def chunk(i, carry):
    base = wid * PW + i * C
    cp_s = pltpu.async_copy(feat.at[nodes_v.at[pl.ds(i * C, C)]], srows, sem_s)
    cp_n = pltpu.async_copy(
        feat.at[neigh_v.at[pl.ds(i * C * NSAMP, C * NSAMP)]], nrows, sem_n)
    cp_l = pltpu.async_copy(
        feat.at[lsh_v.at[pl.ds(i * C * NLSH, C * NLSH)]], lrows, sem_l)
    cp_s.wait()
    pltpu.sync_copy(srows, self_o.at[pl.ds(base, C)])
    cp_n.wait()
    for e in range(C):
      for j in range(D // 16):
        sl = pl.ds(j * 16, 16)
        acc = nrows[e * NSAMP, sl]
        for r in range(1, NSAMP):
          acc = acc + nrows[e * NSAMP + r, sl]
        nmean[e, sl] = acc * jnp.float32(1.0 / NSAMP)
    pltpu.sync_copy(nmean, neigh_o.at[pl.ds(base, C)])
    cp_l.wait()
    for e in range(C):
      for j in range(D // 16):
        sl = pl.ds(j * 16, 16)
        acc = lrows[e * NLSH, sl]
        for r in range(1, NLSH):
          acc = acc + lrows[e * NLSH + r, sl]
        lmean[e, sl] = acc * jnp.float32(1.0 / NLSH)
    pltpu.sync_copy(lmean, lsh_o.at[pl.ds(base, C)])
    return carry

  lax.fori_loop(0, NCHUNK, chunk, 0)


_sc_gather = pl.kernel(
    _sc_body,
    out_type=(
        jax.ShapeDtypeStruct((BP, D), jnp.float32),
        jax.ShapeDtypeStruct((BP, D), jnp.float32),
        jax.ShapeDtypeStruct((BP, D), jnp.float32),
    ),
    mesh=plsc.VectorSubcoreMesh(
        core_axis_name="c", subcore_axis_name="s",
        num_cores=NC, num_subcores=NS),
    scratch_types=[
        pltpu.VMEM((PW,), jnp.int32),
        pltpu.VMEM((PW * NSAMP,), jnp.int32),
        pltpu.VMEM((PW * NLSH,), jnp.int32),
        pltpu.VMEM((C, D), jnp.float32),
        pltpu.VMEM((C * NSAMP, D), jnp.float32),
        pltpu.VMEM((C * NLSH, D), jnp.float32),
        pltpu.VMEM((C, D), jnp.float32),
        pltpu.VMEM((C, D), jnp.float32),
        pltpu.SemaphoreType.DMA,
        pltpu.SemaphoreType.DMA,
        pltpu.SemaphoreType.DMA,
    ],
)


def _mm_body(w_ref, s_ref, n_ref, l_ref, o_ref):
  dn = (((1,), (1,)), ((), ()))
  y = lax.dot_general(w_ref[:, 0:D], s_ref[...], dn,
                      preferred_element_type=jnp.float32)
  y = y + lax.dot_general(w_ref[:, D:2 * D], n_ref[...], dn,
                          preferred_element_type=jnp.float32)
  y = y + lax.dot_general(w_ref[:, 2 * D:3 * D], l_ref[...], dn,
                          preferred_element_type=jnp.float32)
  o_ref[...] = jnp.maximum(y, 0.0)


_matmul = pl.pallas_call(
    _mm_body,
    grid=(BP // TB,),
    in_specs=[
        pl.BlockSpec((E, 3 * D), lambda i: (0, 0)),
        pl.BlockSpec((TB, D), lambda i: (i, 0)),
        pl.BlockSpec((TB, D), lambda i: (i, 0)),
        pl.BlockSpec((TB, D), lambda i: (i, 0)),
    ],
    out_specs=pl.BlockSpec((E, TB), lambda i: (0, i)),
    out_shape=jax.ShapeDtypeStruct((E, B), jnp.float32),
)


@jax.jit
def kernel(nodes, neigh_idx, lsh_idx, features, W):
  pad = BP - B
  nodes_p = jnp.pad(nodes, (0, pad))
  neigh_p = jnp.pad(neigh_idx.reshape(-1), (0, pad * NSAMP))
  lsh_p = jnp.pad(lsh_idx.reshape(-1), (0, pad * NLSH))
  self_f, neigh_f, lsh_f = _sc_gather(features, nodes_p, neigh_p, lsh_p)
  return _matmul(W, self_f, neigh_f, lsh_f)


# speedup vs baseline: 3.1184x; 1.0201x over previous
import jax
import jax.numpy as jnp
from jax import lax
from jax.experimental import pallas as pl
from jax.experimental.pallas import tpu as pltpu
from jax.experimental.pallas import tpu_sc as plsc

B = 50000
D = 128
E = 128
NSAMP = 10
NLSH = 5

NC = 2
NS = 16
NW = NC * NS
C = 16
NCHUNK = 98
PW = C * NCHUNK
BP = NW * PW

TB = 512


def _sc_body(feat, nodes, neigh, lsh, comb_o,
             nodes_v, neigh_v, lsh_v, srows, nrows, lrows, stage,
             gsem, ssem):
  wid = lax.axis_index("s") * NC + lax.axis_index("c")
  pltpu.sync_copy(nodes.at[pl.ds(wid * PW, PW)], nodes_v)
  pltpu.sync_copy(neigh.at[pl.ds(wid * PW * NSAMP, PW * NSAMP)], neigh_v)
  pltpu.sync_copy(lsh.at[pl.ds(wid * PW * NLSH, PW * NLSH)], lsh_v)

  def gather_descs(c, b):
    return (
        pltpu.make_async_copy(
            feat.at[nodes_v.at[pl.ds(c * C, C)]], srows.at[b], gsem.at[b]),
        pltpu.make_async_copy(
            feat.at[neigh_v.at[pl.ds(c * C * NSAMP, C * NSAMP)]],
            nrows.at[b], gsem.at[b]),
        pltpu.make_async_copy(
            feat.at[lsh_v.at[pl.ds(c * C * NLSH, C * NLSH)]],
            lrows.at[b], gsem.at[b]),
    )

  def scatter_desc(c, b):
    base = wid * PW + c * C
    return pltpu.make_async_copy(
        stage.at[b], comb_o.at[pl.ds(base, C)], ssem.at[b])

  for d in gather_descs(0, 0):
    d.start()
  for d in gather_descs(1, 1):
    d.start()

  def body(c, carry):
    b = lax.rem(c, 2)
    for d in gather_descs(c, b):
      d.wait()

    @pl.when(c >= 2)
    def _():
      scatter_desc(c, b).wait()

    for e in range(C):
      for j in range(D // 16):
        sl = pl.ds(j * 16, 16)
        stage[b, e, sl] = srows[b, e, sl]
        accn = nrows[b, e * NSAMP, sl]
        for r in range(1, NSAMP):
          accn = accn + nrows[b, e * NSAMP + r, sl]
        stage[b, e, pl.ds(D + j * 16, 16)] = accn * jnp.float32(1.0 / NSAMP)
        accl = lrows[b, e * NLSH, sl]
        for r in range(1, NLSH):
          accl = accl + lrows[b, e * NLSH + r, sl]
        stage[b, e, pl.ds(2 * D + j * 16, 16)] = accl * jnp.float32(1.0 / NLSH)

    scatter_desc(c, b).start()

    @pl.when(c + 2 < NCHUNK)
    def _():
      for d in gather_descs(c + 2, b):
        d.start()

    return carry

  lax.fori_loop(0, NCHUNK, body, 0)

  scatter_desc(NCHUNK - 2, 0).wait()
  scatter_desc(NCHUNK - 1, 1).wait()


_sc_gather = pl.kernel(
    _sc_body,
    out_type=jax.ShapeDtypeStruct((BP, 3 * D), jnp.float32),
    mesh=plsc.VectorSubcoreMesh(
        core_axis_name="c", subcore_axis_name="s",
        num_cores=NC, num_subcores=NS),
    scratch_types=[
        pltpu.VMEM((PW,), jnp.int32),
        pltpu.VMEM((PW * NSAMP,), jnp.int32),
        pltpu.VMEM((PW * NLSH,), jnp.int32),
        pltpu.VMEM((2, C, D), jnp.float32),
        pltpu.VMEM((2, C * NSAMP, D), jnp.float32),
        pltpu.VMEM((2, C * NLSH, D), jnp.float32),
        pltpu.VMEM((2, C, 3 * D), jnp.float32),
        pltpu.SemaphoreType.DMA((2,)),
        pltpu.SemaphoreType.DMA((2,)),
    ],
)


def _mm_body(w_ref, x_ref, o_ref):
  y = lax.dot_general(w_ref[...], x_ref[...], (((1,), (1,)), ((), ())),
                      preferred_element_type=jnp.float32)
  o_ref[...] = jnp.maximum(y, 0.0)


_matmul = pl.pallas_call(
    _mm_body,
    grid=(BP // TB,),
    in_specs=[
        pl.BlockSpec((E, 3 * D), lambda i: (0, 0)),
        pl.BlockSpec((TB, 3 * D), lambda i: (i, 0)),
    ],
    out_specs=pl.BlockSpec((E, TB), lambda i: (0, i)),
    out_shape=jax.ShapeDtypeStruct((E, B), jnp.float32),
)


@jax.jit
def kernel(nodes, neigh_idx, lsh_idx, features, W):
  pad = BP - B
  nodes_p = jnp.pad(nodes, (0, pad))
  neigh_p = jnp.pad(neigh_idx.reshape(-1), (0, pad * NSAMP))
  lsh_p = jnp.pad(lsh_idx.reshape(-1), (0, pad * NLSH))
  combined = _sc_gather(features, nodes_p, neigh_p, lsh_p)
  return _matmul(W, combined)


# speedup vs baseline: 5.2664x; 1.6888x over previous
import jax
import jax.numpy as jnp
from jax import lax
from jax.experimental import pallas as pl
from jax.experimental.pallas import tpu as pltpu
from jax.experimental.pallas import tpu_sc as plsc

B = 50000
D = 128
E = 128
NSAMP = 10
NLSH = 5

NC = 2
NS = 16
NW = NC * NS
C = 56
NCHUNK = 28
PW = C * NCHUNK
BP = NW * PW

TB = 512


def _sc_body(feat, nodes, neigh, lsh, comb_o,
             nodes_v, neigh_v, lsh_v, srows, nacc, lacc, stage,
             gsem, asem, ssem):
  wid = lax.axis_index("s") * NC + lax.axis_index("c")
  pltpu.sync_copy(nodes.at[pl.ds(wid * PW, PW)], nodes_v)
  for r in range(NSAMP):
    pltpu.sync_copy(neigh.at[pl.ds(r * BP + wid * PW, PW)],
                    neigh_v.at[pl.ds(r * PW, PW)])
  for r in range(NLSH):
    pltpu.sync_copy(lsh.at[pl.ds(r * BP + wid * PW, PW)],
                    lsh_v.at[pl.ds(r * PW, PW)])

  def base_descs(c, b):
    return (
        pltpu.make_async_copy(
            feat.at[nodes_v.at[pl.ds(c * C, C)]], srows.at[b], gsem.at[b]),
        pltpu.make_async_copy(
            feat.at[neigh_v.at[pl.ds(c * C, C)]], nacc.at[b], gsem.at[b]),
        pltpu.make_async_copy(
            feat.at[lsh_v.at[pl.ds(c * C, C)]], lacc.at[b], gsem.at[b]),
    )

  def issue_adds(c, b):
    for r in range(1, NSAMP):
      pltpu.async_copy(
          feat.at[neigh_v.at[pl.ds(r * PW + c * C, C)]], nacc.at[b],
          asem.at[b], add=True)
    for r in range(1, NLSH):
      pltpu.async_copy(
          feat.at[lsh_v.at[pl.ds(r * PW + c * C, C)]], lacc.at[b],
          asem.at[b], add=True)

  def wait_adds(c, b):
    d = pltpu.make_async_copy(
        feat.at[neigh_v.at[pl.ds(c * C, C)]], nacc.at[b], asem.at[b])
    for _ in range(NSAMP - 1 + NLSH - 1):
      d.wait()

  def scatter_desc(c, b):
    base = wid * PW + c * C
    return pltpu.make_async_copy(
        stage.at[b], comb_o.at[pl.ds(base, C)], ssem.at[b])

  for d in base_descs(0, 0):
    d.start()
  for d in base_descs(1, 1):
    d.start()
  for d in base_descs(0, 0):
    d.wait()
  issue_adds(0, 0)

  def body(c, carry):
    b = lax.rem(c, 2)
    nb = 1 - b

    @pl.when(c + 1 < NCHUNK)
    def _():
      for d in base_descs(c + 1, nb):
        d.wait()
      issue_adds(c + 1, nb)

    wait_adds(c, b)

    @pl.when(c >= 2)
    def _():
      scatter_desc(c, b).wait()

    for e in range(C):
      for j in range(D // 16):
        sl = pl.ds(j * 16, 16)
        stage[b, e, sl] = srows[b, e, sl]
        stage[b, e, pl.ds(D + j * 16, 16)] = (
            nacc[b, e, sl] * jnp.float32(1.0 / NSAMP))
        stage[b, e, pl.ds(2 * D + j * 16, 16)] = (
            lacc[b, e, sl] * jnp.float32(1.0 / NLSH))

    scatter_desc(c, b).start()

    @pl.when(c + 2 < NCHUNK)
    def _():
      for d in base_descs(c + 2, b):
        d.start()

    return carry

  lax.fori_loop(0, NCHUNK, body, 0)

  scatter_desc(NCHUNK - 2, 0).wait()
  scatter_desc(NCHUNK - 1, 1).wait()


_sc_gather = pl.kernel(
    _sc_body,
    out_type=jax.ShapeDtypeStruct((BP, 3 * D), jnp.float32),
    mesh=plsc.VectorSubcoreMesh(
        core_axis_name="c", subcore_axis_name="s",
        num_cores=NC, num_subcores=NS),
    scratch_types=[
        pltpu.VMEM((PW,), jnp.int32),
        pltpu.VMEM((PW * NSAMP,), jnp.int32),
        pltpu.VMEM((PW * NLSH,), jnp.int32),
        pltpu.VMEM((2, C, D), jnp.float32),
        pltpu.VMEM((2, C, D), jnp.float32),
        pltpu.VMEM((2, C, D), jnp.float32),
        pltpu.VMEM((2, C, 3 * D), jnp.float32),
        pltpu.SemaphoreType.DMA((2,)),
        pltpu.SemaphoreType.DMA((2,)),
        pltpu.SemaphoreType.DMA((2,)),
    ],
)


def _mm_body(w_ref, x_ref, o_ref):
  y = lax.dot_general(w_ref[...], x_ref[...], (((1,), (1,)), ((), ())),
                      preferred_element_type=jnp.float32)
  o_ref[...] = jnp.maximum(y, 0.0)


_matmul = pl.pallas_call(
    _mm_body,
    grid=(BP // TB,),
    in_specs=[
        pl.BlockSpec((E, 3 * D), lambda i: (0, 0)),
        pl.BlockSpec((TB, 3 * D), lambda i: (i, 0)),
    ],
    out_specs=pl.BlockSpec((E, TB), lambda i: (0, i)),
    out_shape=jax.ShapeDtypeStruct((E, B), jnp.float32),
)


@jax.jit
def kernel(nodes, neigh_idx, lsh_idx, features, W):
  pad = BP - B
  nodes_p = jnp.pad(nodes, (0, pad))
  neigh_p = jnp.pad(neigh_idx, ((0, pad), (0, 0))).T.reshape(-1)
  lsh_p = jnp.pad(lsh_idx, ((0, pad), (0, 0))).T.reshape(-1)
  combined = _sc_gather(features, nodes_p, neigh_p, lsh_p)
  return _matmul(W, combined)
